# baseline (speedup 1.0000x reference)
import functools

import jax
import jax.numpy as jnp
from jax import lax
from jax.experimental import pallas as pl
from jax.experimental.pallas import tpu as pltpu
from jax.experimental.pallas import tpu_sc as plsc

N0 = 10000
D = 128
NC = 2
NS = 16
NW = NC * NS
CHUNK = 128
CPT = 80
EPT = CHUNK * CPT
E_PAD = NW * EPT
NROW_PAD = 10240
NPC = NROW_PAD // NC
ACC_ROWS = 6144
SCAT_PAD = 1 << 30
NBUF = 4
BE = 512
BN = 1000


def _mesh():
    return plsc.VectorSubcoreMesh(core_axis_name="c", subcore_axis_name="s",
                                  num_cores=NC, num_subcores=NS)


def _make_gather():
    @functools.partial(
        pl.kernel,
        out_type=jax.ShapeDtypeStruct((NW, CPT // NBUF, NBUF * CHUNK, D), jnp.float32),
        mesh=_mesh(),
        scratch_types=[
            pltpu.VMEM((CPT, CHUNK), jnp.int32),
            pltpu.VMEM((NBUF * CHUNK, D), jnp.float32),
            pltpu.SemaphoreType.DMA,
            pltpu.SemaphoreType.DMA,
            pltpu.SemaphoreType.DMA,
            pltpu.SemaphoreType.DMA,
        ],
    )
    def gather_k(tbl, idx, out, idx_v, wbuf, sem0, sem1, sem2, sem3):
        sems = [sem0, sem1, sem2, sem3]
        c = lax.axis_index("c")
        s = lax.axis_index("s")
        w = c * NS + s
        pltpu.sync_copy(idx.at[w], idx_v)

        def outer(t, _):
            cps = []
            for b in range(NBUF):
                cps.append(pltpu.async_copy(
                    tbl.at[idx_v.at[t * NBUF + b]],
                    wbuf.at[pl.ds(b * CHUNK, CHUNK)], sems[b]))
            for cp in cps:
                cp.wait()
            pltpu.sync_copy(wbuf, out.at[w, t])
            return _

        lax.fori_loop(0, CPT // NBUF, outer, None)

    return gather_k


def _make_gather3():
    OUT3 = jax.ShapeDtypeStruct((NW, CPT // NBUF, NBUF * CHUNK, D), jnp.float32)

    @functools.partial(
        pl.kernel,
        out_type=(OUT3, OUT3, OUT3),
        mesh=_mesh(),
        scratch_types=[
            pltpu.VMEM((CPT, CHUNK), jnp.int32),
            pltpu.VMEM((CPT, CHUNK), jnp.int32),
            pltpu.VMEM((NBUF * CHUNK, D), jnp.float32),
            pltpu.SemaphoreType.DMA,
            pltpu.SemaphoreType.DMA,
            pltpu.SemaphoreType.DMA,
            pltpu.SemaphoreType.DMA,
        ],
    )
    def gather3_k(tbl_a, tbl_b, tbl_c, idx_r, idx_c, out_a, out_b, out_c,
                  idxr_v, idxc_v, wbuf, sem0, sem1, sem2, sem3):
        sems = [sem0, sem1, sem2, sem3]
        c = lax.axis_index("c")
        s = lax.axis_index("s")
        w = c * NS + s
        pltpu.sync_copy(idx_r.at[w], idxr_v)
        pltpu.sync_copy(idx_c.at[w], idxc_v)

        for tbl, idx_v, out in ((tbl_a, idxr_v, out_a),
                                (tbl_b, idxc_v, out_b),
                                (tbl_c, idxc_v, out_c)):
            def outer(t, _):
                cps = []
                for b in range(NBUF):
                    cps.append(pltpu.async_copy(
                        tbl.at[idx_v.at[t * NBUF + b]],
                        wbuf.at[pl.ds(b * CHUNK, CHUNK)], sems[b]))
                for cp in cps:
                    cp.wait()
                pltpu.sync_copy(wbuf, out.at[w, t])
                return _

            lax.fori_loop(0, CPT // NBUF, outer, None)

    return gather3_k


def _make_scatter():

    @functools.partial(
        pl.kernel,
        out_type=jax.ShapeDtypeStruct((NROW_PAD, D), jnp.float32),
        mesh=_mesh(),
        scratch_types=[
            pltpu.VMEM((CPT, CHUNK), jnp.int32),
            pltpu.VMEM((CPT, CHUNK), jnp.int32),
            pltpu.VMEM((2, CHUNK, D), jnp.float32),
            pltpu.VMEM((CHUNK, D), jnp.float32),
            pltpu.VMEM_SHARED((ACC_ROWS, D), jnp.float32),
            pltpu.SemaphoreType.DMA,
            pltpu.SemaphoreType.DMA,
        ],
    )
    def scatter_k(ea, ridx, zeros_b, out_s, idx_v, lidx_v, dbuf, zbuf, acc,
                  sem0, sem1):
        c = lax.axis_index("c")
        s = lax.axis_index("s")
        pltpu.sync_copy(zeros_b, zbuf)

        zpt = ACC_ROWS // NS // CHUNK
        z0 = s * (ACC_ROWS // NS)

        def zbody(k, _):
            pltpu.sync_copy(zbuf, acc.at[pl.ds(z0 + k * CHUNK, CHUNK)])
            return _

        lax.fori_loop(0, zpt, zbody, None)
        plsc.subcore_barrier()

        base = c * NPC

        def do_partition(p):
            pltpu.sync_copy(ridx.at[p], idx_v)

            def remap(g, _):
                r = g >> 3
                k = g & 7
                raw = idx_v[r, pl.ds(16 * k, 16)]
                loc = raw - base
                ok = (loc >= 0) & (loc < NPC)
                lidx_v[r, pl.ds(16 * k, 16)] = jnp.where(ok, loc, ACC_ROWS - 1)
                return _

            lax.fori_loop(0, CPT * (CHUNK // 16), remap, None)

            sems = [sem0, sem1]
            pltpu.async_copy(ea.at[p, 0], dbuf.at[0], sem0).wait()

            def body(t, _):
                for par in range(2):
                    j = 2 * t + par
                    nxt = 1 - par

                    @pl.when(j + 1 < CPT)
                    def _():
                        pltpu.async_copy(ea.at[p, j + 1], dbuf.at[nxt],
                                         sems[nxt])
                    pltpu.sync_copy(dbuf.at[par], acc.at[lidx_v.at[j]],
                                    add=True)

                    @pl.when(j + 1 < CPT)
                    def _():
                        pltpu.make_async_copy(ea.at[p, j + 1], dbuf.at[nxt],
                                              sems[nxt]).wait()
                return _

            lax.fori_loop(0, CPT // 2, body, None)

        do_partition(2 * s)
        do_partition(2 * s + 1)
        plsc.subcore_barrier()

        w0 = (NPC // NS) * s

        def wbody(k, _):
            rr = pl.ds(w0 + k * 64, 64)
            pltpu.sync_copy(acc.at[rr], zbuf.at[pl.ds(0, 64)])
            pltpu.sync_copy(zbuf.at[pl.ds(0, 64)], out_s.at[pl.ds(base + w0 + k * 64, 64)])
            return _

        lax.fori_loop(0, NPC // NS // 64, wbody, None)

    return scatter_k


_SC_CACHE = {}


def _sc_gather():
    if 'g' not in _SC_CACHE:
        _SC_CACHE['g'] = _make_gather()
    return _SC_CACHE['g']


def _sc_scatter():
    if 's' not in _SC_CACHE:
        _SC_CACHE['s'] = _make_scatter()
    return _SC_CACHE['s']


def _sc_gather3():
    if 'g3' not in _SC_CACHE:
        _SC_CACHE['g3'] = _make_gather3()
    return _SC_CACHE['g3']


def _proj_body(x0_r, x1_r, ew1a, ew1b, eb1, pw1b, pb1, u0, u1, v1):
    u0[...] = jnp.dot(x0_r[...], ew1a[...], preferred_element_type=jnp.float32)
    u1[...] = jnp.dot(x1_r[...], ew1b[...], preferred_element_type=jnp.float32) + eb1[...]
    v1[...] = jnp.dot(x1_r[...], pw1b[...], preferred_element_type=jnp.float32) + pb1[...]


def _mean_var(h):
    oc = jnp.full((D, 1), 1.0 / D, jnp.float32)
    m = jnp.dot(h, oc, preferred_element_type=jnp.float32)
    msq = jnp.dot(h * h, oc, preferred_element_type=jnp.float32)
    return m, msq - m * m


def _edge_body(a_r, b_r, w2, b2, g, be, w3, b3, out_r):
    h1 = jnp.maximum(a_r[...] + b_r[...], 0.0)
    h2 = jnp.dot(h1, w2[...], preferred_element_type=jnp.float32) + b2[...]
    m, v = _mean_var(h2)
    y = (h2 - m) * lax.rsqrt(v + 1e-5) * g[...] + be[...]
    y = jnp.maximum(y, 0.0)
    out_r[...] = jnp.dot(y, w3[...], preferred_element_type=jnp.float32) + b3[...]


def _node_body(s_r, x0_r, nw1a, nw1b, nb1, nw2, nb2, ng, nbe,
               nw3, nb3, pw1a, v0_r):
    agg = s_r[...]
    x = x0_r[...]
    h = jnp.maximum(
        jnp.dot(x, nw1a[...], preferred_element_type=jnp.float32)
        + jnp.dot(agg, nw1b[...], preferred_element_type=jnp.float32)
        + nb1[...], 0.0)
    h = jnp.dot(h, nw2[...], preferred_element_type=jnp.float32) + nb2[...]
    m, v = _mean_var(h)
    h = jnp.maximum((h - m) * lax.rsqrt(v + 1e-5) * ng[...] + nbe[...], 0.0)
    na = jnp.dot(h, nw3[...], preferred_element_type=jnp.float32) + nb3[...] + x
    v0_r[...] = jnp.dot(na, pw1a[...], preferred_element_type=jnp.float32)


def _pred_body(a_r, b_r, w2, b2, out_r):
    t = jnp.maximum(a_r[...] + b_r[...], 0.0)
    s = jnp.dot(t, w2[...], preferred_element_type=jnp.float32) + b2[...]
    out_r[...] = jax.nn.sigmoid(s)


def _wspec(shape):
    return pl.BlockSpec(shape, lambda i: tuple(0 for _ in shape))


def kernel(x0, x1, edge_index, e_w1, e_b1, e_w2, e_b2, e_g, e_be, e_w3, e_b3,
           n_w1, n_b1, n_w2, n_b2, n_g, n_be, n_w3, n_b3,
           p_w1, p_b1, p_w2, p_b2):
    E = edge_index.shape[1]
    row = edge_index[0]
    col = edge_index[1]
    npad = E_PAD - E
    row_g = jnp.concatenate([row, jnp.zeros((npad,), jnp.int32)]).reshape(NW, CPT, CHUNK)
    col_g = jnp.concatenate([col, jnp.zeros((npad,), jnp.int32)]).reshape(NW, CPT, CHUNK)
    row_s = jnp.concatenate([row, jnp.full((npad,), SCAT_PAD, jnp.int32)]).reshape(NW, CPT, CHUNK)

    nb = N0 // BN
    proj = pl.pallas_call(
        _proj_body,
        grid=(nb,),
        in_specs=[
            pl.BlockSpec((BN, D), lambda i: (i, 0)),
            pl.BlockSpec((BN, D), lambda i: (i, 0)),
            _wspec((D, D)), _wspec((D, D)), _wspec((1, D)),
            _wspec((D, D)), _wspec((1, D)),
        ],
        out_specs=[pl.BlockSpec((BN, D), lambda i: (i, 0))] * 3,
        out_shape=[jax.ShapeDtypeStruct((N0, D), jnp.float32)] * 3,
    )
    u0, u1, v1 = proj(x0, x1, e_w1[:D], e_w1[D:], e_b1.reshape(1, D),
                      p_w1[D:], p_b1.reshape(1, D))

    g_u0 = _sc_gather()(u0, row_g).reshape(E_PAD, D)
    g_u1 = _sc_gather()(u1, col_g).reshape(E_PAD, D)
    g_v1 = _sc_gather()(v1, col_g).reshape(E_PAD, D)

    ne = E_PAD // BE
    ea = pl.pallas_call(
        _edge_body,
        grid=(ne,),
        in_specs=[
            pl.BlockSpec((BE, D), lambda i: (i, 0)),
            pl.BlockSpec((BE, D), lambda i: (i, 0)),
            _wspec((D, D)), _wspec((1, D)), _wspec((1, D)), _wspec((1, D)),
            _wspec((D, D)), _wspec((1, D)),
        ],
        out_specs=pl.BlockSpec((BE, D), lambda i: (i, 0)),
        out_shape=jax.ShapeDtypeStruct((E_PAD, D), jnp.float32),
    )(g_u0, g_u1, e_w2, e_b2.reshape(1, D), e_g.reshape(1, D), e_be.reshape(1, D),
      e_w3, e_b3.reshape(1, D))

    zeros_b = jnp.zeros((CHUNK, D), jnp.float32)
    s_sum = _sc_scatter()(ea.reshape(NW, CPT, CHUNK, D), row_s, zeros_b)

    v0 = pl.pallas_call(
        _node_body,
        grid=(nb,),
        in_specs=[
            pl.BlockSpec((BN, D), lambda i: (i, 0)),
            pl.BlockSpec((BN, D), lambda i: (i, 0)),
            _wspec((D, D)), _wspec((D, D)), _wspec((1, D)),
            _wspec((D, D)), _wspec((1, D)), _wspec((1, D)), _wspec((1, D)),
            _wspec((D, D)), _wspec((1, D)), _wspec((D, D)),
        ],
        out_specs=pl.BlockSpec((BN, D), lambda i: (i, 0)),
        out_shape=jax.ShapeDtypeStruct((N0, D), jnp.float32),
    )(s_sum, x0,
      n_w1[:D], n_w1[D:], n_b1.reshape(1, D), n_w2, n_b2.reshape(1, D),
      n_g.reshape(1, D), n_be.reshape(1, D), n_w3, n_b3.reshape(1, D), p_w1[:D])

    g_v0 = _sc_gather()(v0, row_g).reshape(E_PAD, D)

    preds = pl.pallas_call(
        _pred_body,
        grid=(ne,),
        in_specs=[
            pl.BlockSpec((BE, D), lambda i: (i, 0)),
            pl.BlockSpec((BE, D), lambda i: (i, 0)),
            _wspec((D, 1)), _wspec((1, 1)),
        ],
        out_specs=pl.BlockSpec((BE, 1), lambda i: (i, 0)),
        out_shape=jax.ShapeDtypeStruct((E_PAD, 1), jnp.float32),
    )(g_v0, g_v1, p_w2, p_b2.reshape(1, 1))

    return preds[:E]


# speedup vs baseline: 1.4426x; 1.4426x over previous
import functools

import jax
import jax.numpy as jnp
from jax import lax
from jax.experimental import pallas as pl
from jax.experimental.pallas import tpu as pltpu
from jax.experimental.pallas import tpu_sc as plsc

N0 = 10000
D = 128
NC = 2
NS = 16
NW = NC * NS
CHUNK = 128
CPT = 80
EPT = CHUNK * CPT
E_PAD = NW * EPT
NROW_PAD = 10240
NPC = NROW_PAD // NC
ACC_ROWS = 6144
SCAT_PAD = 1 << 30
NBUF = 4
BE = 8192
BN = 2000


def _mesh():
    return plsc.VectorSubcoreMesh(core_axis_name="c", subcore_axis_name="s",
                                  num_cores=NC, num_subcores=NS)


def _make_gather():
    @functools.partial(
        pl.kernel,
        out_type=jax.ShapeDtypeStruct((NW, CPT // NBUF, NBUF * CHUNK, D), jnp.float32),
        mesh=_mesh(),
        scratch_types=[
            pltpu.VMEM((CPT, CHUNK), jnp.int32),
            pltpu.VMEM((NBUF * CHUNK, D), jnp.float32),
            pltpu.SemaphoreType.DMA,
            pltpu.SemaphoreType.DMA,
            pltpu.SemaphoreType.DMA,
            pltpu.SemaphoreType.DMA,
        ],
    )
    def gather_k(tbl, idx, out, idx_v, wbuf, sem0, sem1, sem2, sem3):
        sems = [sem0, sem1, sem2, sem3]
        c = lax.axis_index("c")
        s = lax.axis_index("s")
        w = c * NS + s
        pltpu.sync_copy(idx.at[w], idx_v)

        def outer(t, _):
            cps = []
            for b in range(NBUF):
                cps.append(pltpu.async_copy(
                    tbl.at[idx_v.at[t * NBUF + b]],
                    wbuf.at[pl.ds(b * CHUNK, CHUNK)], sems[b]))
            for cp in cps:
                cp.wait()
            pltpu.sync_copy(wbuf, out.at[w, t])
            return _

        lax.fori_loop(0, CPT // NBUF, outer, None)

    return gather_k


def _make_gather3():
    OUT3 = jax.ShapeDtypeStruct((NW, CPT // NBUF, NBUF * CHUNK, D), jnp.float32)

    @functools.partial(
        pl.kernel,
        out_type=(OUT3, OUT3, OUT3),
        mesh=_mesh(),
        scratch_types=[
            pltpu.VMEM((CPT, CHUNK), jnp.int32),
            pltpu.VMEM((CPT, CHUNK), jnp.int32),
            pltpu.VMEM((NBUF * CHUNK, D), jnp.float32),
            pltpu.SemaphoreType.DMA,
            pltpu.SemaphoreType.DMA,
            pltpu.SemaphoreType.DMA,
            pltpu.SemaphoreType.DMA,
        ],
    )
    def gather3_k(tbl_a, tbl_b, tbl_c, idx_r, idx_c, out_a, out_b, out_c,
                  idxr_v, idxc_v, wbuf, sem0, sem1, sem2, sem3):
        sems = [sem0, sem1, sem2, sem3]
        c = lax.axis_index("c")
        s = lax.axis_index("s")
        w = c * NS + s
        pltpu.sync_copy(idx_r.at[w], idxr_v)
        pltpu.sync_copy(idx_c.at[w], idxc_v)

        for tbl, idx_v, out in ((tbl_a, idxr_v, out_a),
                                (tbl_b, idxc_v, out_b),
                                (tbl_c, idxc_v, out_c)):
            def outer(t, _):
                cps = []
                for b in range(NBUF):
                    cps.append(pltpu.async_copy(
                        tbl.at[idx_v.at[t * NBUF + b]],
                        wbuf.at[pl.ds(b * CHUNK, CHUNK)], sems[b]))
                for cp in cps:
                    cp.wait()
                pltpu.sync_copy(wbuf, out.at[w, t])
                return _

            lax.fori_loop(0, CPT // NBUF, outer, None)

    return gather3_k


def _make_scatter():

    @functools.partial(
        pl.kernel,
        out_type=jax.ShapeDtypeStruct((NROW_PAD, D), jnp.float32),
        mesh=_mesh(),
        scratch_types=[
            pltpu.VMEM((CPT, CHUNK), jnp.int32),
            pltpu.VMEM((CPT, CHUNK), jnp.int32),
            pltpu.VMEM((2, CHUNK, D), jnp.float32),
            pltpu.VMEM((CHUNK, D), jnp.float32),
            pltpu.VMEM_SHARED((ACC_ROWS, D), jnp.float32),
            pltpu.SemaphoreType.DMA,
            pltpu.SemaphoreType.DMA,
        ],
    )
    def scatter_k(ea, ridx, zeros_b, out_s, idx_v, lidx_v, dbuf, zbuf, acc,
                  sem0, sem1):
        c = lax.axis_index("c")
        s = lax.axis_index("s")
        pltpu.sync_copy(zeros_b, zbuf)

        zpt = ACC_ROWS // NS // CHUNK
        z0 = s * (ACC_ROWS // NS)

        def zbody(k, _):
            pltpu.sync_copy(zbuf, acc.at[pl.ds(z0 + k * CHUNK, CHUNK)])
            return _

        lax.fori_loop(0, zpt, zbody, None)
        plsc.subcore_barrier()

        base = c * NPC

        def do_partition(p):
            pltpu.sync_copy(ridx.at[p], idx_v)

            def remap(g, _):
                r = g >> 3
                k = g & 7
                raw = idx_v[r, pl.ds(16 * k, 16)]
                loc = raw - base
                ok = (loc >= 0) & (loc < NPC)
                lidx_v[r, pl.ds(16 * k, 16)] = jnp.where(ok, loc, ACC_ROWS - 1)
                return _

            lax.fori_loop(0, CPT * (CHUNK // 16), remap, None)

            sems = [sem0, sem1]
            pltpu.async_copy(ea.at[p, 0], dbuf.at[0], sem0).wait()

            def body(t, _):
                for par in range(2):
                    j = 2 * t + par
                    nxt = 1 - par

                    @pl.when(j + 1 < CPT)
                    def _():
                        pltpu.async_copy(ea.at[p, j + 1], dbuf.at[nxt],
                                         sems[nxt])
                    pltpu.sync_copy(dbuf.at[par], acc.at[lidx_v.at[j]],
                                    add=True)

                    @pl.when(j + 1 < CPT)
                    def _():
                        pltpu.make_async_copy(ea.at[p, j + 1], dbuf.at[nxt],
                                              sems[nxt]).wait()
                return _

            lax.fori_loop(0, CPT // 2, body, None)

        do_partition(2 * s)
        do_partition(2 * s + 1)
        plsc.subcore_barrier()

        w0 = (NPC // NS) * s

        def wbody(k, _):
            rr = pl.ds(w0 + k * 64, 64)
            pltpu.sync_copy(acc.at[rr], zbuf.at[pl.ds(0, 64)])
            pltpu.sync_copy(zbuf.at[pl.ds(0, 64)], out_s.at[pl.ds(base + w0 + k * 64, 64)])
            return _

        lax.fori_loop(0, NPC // NS // 64, wbody, None)

    return scatter_k


_SC_CACHE = {}


def _sc_gather():
    if 'g' not in _SC_CACHE:
        _SC_CACHE['g'] = _make_gather()
    return _SC_CACHE['g']


def _sc_scatter():
    if 's' not in _SC_CACHE:
        _SC_CACHE['s'] = _make_scatter()
    return _SC_CACHE['s']


def _sc_gather3():
    if 'g3' not in _SC_CACHE:
        _SC_CACHE['g3'] = _make_gather3()
    return _SC_CACHE['g3']


def _proj_body(x0_r, x1_r, ew1a, ew1b, eb1, pw1b, pb1, u0, u1, v1):
    u0[...] = jnp.dot(x0_r[...], ew1a[...], preferred_element_type=jnp.float32)
    u1[...] = jnp.dot(x1_r[...], ew1b[...], preferred_element_type=jnp.float32) + eb1[...]
    v1[...] = jnp.dot(x1_r[...], pw1b[...], preferred_element_type=jnp.float32) + pb1[...]


def _edge_body(a_r, b_r, w2, b2, g, be, w3, b3, out_r):
    h1 = jnp.maximum(a_r[...] + b_r[...], 0.0)
    h2 = jnp.dot(h1, w2[...], preferred_element_type=jnp.float32) + b2[...]
    m = jnp.mean(h2, axis=-1, keepdims=True)
    d = h2 - m
    v = jnp.mean(d * d, axis=-1, keepdims=True)
    y = jnp.maximum(d * lax.rsqrt(v + 1e-5) * g[...] + be[...], 0.0)
    out_r[...] = jnp.dot(y, w3[...], preferred_element_type=jnp.float32) + b3[...]


def _node_body(s_r, x0_r, nw1a, nw1b, nb1, nw2, nb2, ng, nbe,
               nw3, nb3, pw1a, v0_r):
    agg = s_r[...]
    x = x0_r[...]
    h = jnp.maximum(
        jnp.dot(x, nw1a[...], preferred_element_type=jnp.float32)
        + jnp.dot(agg, nw1b[...], preferred_element_type=jnp.float32)
        + nb1[...], 0.0)
    h = jnp.dot(h, nw2[...], preferred_element_type=jnp.float32) + nb2[...]
    m = jnp.mean(h, axis=-1, keepdims=True)
    d = h - m
    v = jnp.mean(d * d, axis=-1, keepdims=True)
    h = jnp.maximum(d * lax.rsqrt(v + 1e-5) * ng[...] + nbe[...], 0.0)
    na = jnp.dot(h, nw3[...], preferred_element_type=jnp.float32) + nb3[...] + x
    v0_r[...] = jnp.dot(na, pw1a[...], preferred_element_type=jnp.float32)


def _pred_body(a_r, b_r, w2, b2, out_r):
    t = jnp.maximum(a_r[...] + b_r[...], 0.0)
    s = jnp.sum(t * w2[...], axis=1, keepdims=True) + b2[...]
    out_r[...] = jax.nn.sigmoid(s)


def _wspec(shape):
    return pl.BlockSpec(shape, lambda i: tuple(0 for _ in shape))


def kernel(x0, x1, edge_index, e_w1, e_b1, e_w2, e_b2, e_g, e_be, e_w3, e_b3,
           n_w1, n_b1, n_w2, n_b2, n_g, n_be, n_w3, n_b3,
           p_w1, p_b1, p_w2, p_b2):
    E = edge_index.shape[1]
    row = edge_index[0]
    col = edge_index[1]
    npad = E_PAD - E
    row_g = jnp.concatenate([row, jnp.zeros((npad,), jnp.int32)]).reshape(NW, CPT, CHUNK)
    col_g = jnp.concatenate([col, jnp.zeros((npad,), jnp.int32)]).reshape(NW, CPT, CHUNK)
    row_s = jnp.concatenate([row, jnp.full((npad,), SCAT_PAD, jnp.int32)]).reshape(NW, CPT, CHUNK)

    nb = N0 // BN
    proj = pl.pallas_call(
        _proj_body,
        grid=(nb,),
        in_specs=[
            pl.BlockSpec((BN, D), lambda i: (i, 0)),
            pl.BlockSpec((BN, D), lambda i: (i, 0)),
            _wspec((D, D)), _wspec((D, D)), _wspec((1, D)),
            _wspec((D, D)), _wspec((1, D)),
        ],
        out_specs=[pl.BlockSpec((BN, D), lambda i: (i, 0))] * 3,
        out_shape=[jax.ShapeDtypeStruct((N0, D), jnp.float32)] * 3,
    )
    u0, u1, v1 = proj(x0, x1, e_w1[:D], e_w1[D:], e_b1.reshape(1, D),
                      p_w1[D:], p_b1.reshape(1, D))

    g_u0 = _sc_gather()(u0, row_g).reshape(E_PAD, D)
    g_u1 = _sc_gather()(u1, col_g).reshape(E_PAD, D)
    g_v1 = _sc_gather()(v1, col_g).reshape(E_PAD, D)

    ne = E_PAD // BE
    ea = pl.pallas_call(
        _edge_body,
        grid=(ne,),
        in_specs=[
            pl.BlockSpec((BE, D), lambda i: (i, 0)),
            pl.BlockSpec((BE, D), lambda i: (i, 0)),
            _wspec((D, D)), _wspec((1, D)), _wspec((1, D)), _wspec((1, D)),
            _wspec((D, D)), _wspec((1, D)),
        ],
        out_specs=pl.BlockSpec((BE, D), lambda i: (i, 0)),
        out_shape=jax.ShapeDtypeStruct((E_PAD, D), jnp.float32),
    )(g_u0, g_u1, e_w2, e_b2.reshape(1, D), e_g.reshape(1, D), e_be.reshape(1, D),
      e_w3, e_b3.reshape(1, D))

    zeros_b = jnp.zeros((CHUNK, D), jnp.float32)
    s_sum = _sc_scatter()(ea.reshape(NW, CPT, CHUNK, D), row_s, zeros_b)

    v0 = pl.pallas_call(
        _node_body,
        grid=(nb,),
        in_specs=[
            pl.BlockSpec((BN, D), lambda i: (i, 0)),
            pl.BlockSpec((BN, D), lambda i: (i, 0)),
            _wspec((D, D)), _wspec((D, D)), _wspec((1, D)),
            _wspec((D, D)), _wspec((1, D)), _wspec((1, D)), _wspec((1, D)),
            _wspec((D, D)), _wspec((1, D)), _wspec((D, D)),
        ],
        out_specs=pl.BlockSpec((BN, D), lambda i: (i, 0)),
        out_shape=jax.ShapeDtypeStruct((N0, D), jnp.float32),
    )(s_sum, x0,
      n_w1[:D], n_w1[D:], n_b1.reshape(1, D), n_w2, n_b2.reshape(1, D),
      n_g.reshape(1, D), n_be.reshape(1, D), n_w3, n_b3.reshape(1, D), p_w1[:D])

    g_v0 = _sc_gather()(v0, row_g).reshape(E_PAD, D)

    preds = pl.pallas_call(
        _pred_body,
        grid=(ne,),
        in_specs=[
            pl.BlockSpec((BE, D), lambda i: (i, 0)),
            pl.BlockSpec((BE, D), lambda i: (i, 0)),
            _wspec((1, D)), _wspec((1, 1)),
        ],
        out_specs=pl.BlockSpec((BE, 1), lambda i: (i, 0)),
        out_shape=jax.ShapeDtypeStruct((E_PAD, 1), jnp.float32),
    )(g_v0, g_v1, p_w2.reshape(1, D), p_b2.reshape(1, 1))

    return preds[:E]
